# baseline (speedup 1.0000x reference)
import functools
import math

import jax
import jax.numpy as jnp
from jax import lax
from jax.experimental import pallas as pl
from jax.experimental.pallas import tpu as pltpu
from jax.experimental.pallas import tpu_sc as plsc

NUM_CLASSES = 128
LOG_2PI = math.log(2.0 * math.pi)

NC = 2
NS = 16
CH = 64


def _pre_body(t_ref, p_ref, m_ref, pm_out, s_out, idx_out):
    p = p_ref[0]
    m = m_ref[0]
    n_ex, _ = p.shape
    pm = p * m
    pm_out[0] = pm
    r_pmm = jnp.sum(pm * m, axis=1, keepdims=True)
    r_lp = jnp.sum(jnp.log(p), axis=1, keepdims=True)
    ones = jnp.ones((n_ex, 1), jnp.float32)
    zeros = jnp.zeros((n_ex, 13), jnp.float32)
    s_out[0] = jnp.concatenate([r_pmm, r_lp, ones, zeros], axis=1)
    idx_out[0] = t_ref[0] + pl.program_id(0) * NUM_CLASSES


def _sc_body(p_hbm, pm_hbm, s_hbm, idx_hbm, z_hbm, zs_hbm,
             segp_hbm, segpm_hbm, segs_hbm,
             pbuf0, pbuf1, pmbuf0, pmbuf1, sbuf0, sbuf1, idx0, idx1,
             accp, accpm, accs, sem):
    cid = lax.axis_index("c")
    sid = lax.axis_index("s")
    n_rows = segp_hbm.shape[0]
    rows_t = n_rows // NS
    d = p_hbm.shape[1]
    dq = d // (2 * NC)
    ep_t = p_hbm.shape[0] // NS
    nchunk = ep_t // CH
    r0 = sid * rows_t

    pb = (pbuf0, pbuf1)
    pmb = (pmbuf0, pmbuf1)
    sb = (sbuf0, sbuf1)
    ib = (idx0, idx1)

    for q in range(2):
        c0 = (q * NC + cid) * dq
        first = q == 0

        pltpu.sync_copy(z_hbm, accp.at[pl.ds(r0, rows_t)])
        pltpu.sync_copy(z_hbm, accpm.at[pl.ds(r0, rows_t)])
        if first:
            @pl.when(cid == 0)
            def _():
                pltpu.sync_copy(zs_hbm, accs.at[pl.ds(r0, rows_t)])

        plsc.subcore_barrier()

        def fire(g, slot, first=first):
            e0 = sid * ep_t + g * CH
            cps = [
                pltpu.async_copy(idx_hbm.at[pl.ds(e0, CH)], ib[slot], sem),
                pltpu.async_copy(
                    p_hbm.at[pl.ds(e0, CH), pl.ds(c0, dq)], pb[slot], sem),
                pltpu.async_copy(
                    pm_hbm.at[pl.ds(e0, CH), pl.ds(c0, dq)], pmb[slot], sem),
            ]
            if first:
                cps.append(
                    pltpu.async_copy(s_hbm.at[pl.ds(e0, CH)], sb[slot], sem))
            return cps

        pend = fire(0, 0)
        for g in range(nchunk):
            slot = g % 2
            for cp in pend:
                cp.wait()
            if g + 1 < nchunk:
                pend = fire(g + 1, (g + 1) % 2)
            pltpu.sync_copy(pb[slot], accp.at[ib[slot]], add=True)
            pltpu.sync_copy(pmb[slot], accpm.at[ib[slot]], add=True)
            if first:
                @pl.when(cid == 0)
                def _(slot=slot):
                    pltpu.sync_copy(sb[slot], accs.at[ib[slot]], add=True)

        plsc.subcore_barrier()

        pltpu.sync_copy(accp.at[pl.ds(r0, rows_t)],
                        segp_hbm.at[pl.ds(r0, rows_t), pl.ds(c0, dq)])
        pltpu.sync_copy(accpm.at[pl.ds(r0, rows_t)],
                        segpm_hbm.at[pl.ds(r0, rows_t), pl.ds(c0, dq)])
        if first:
            @pl.when(cid == 0)
            def _():
                pltpu.sync_copy(accs.at[pl.ds(r0, rows_t)],
                                segs_hbm.at[pl.ds(r0, rows_t)])


def _fin_body(segp_ref, segpm_ref, segs_ref, pm_out, ln_out):
    sp = segp_ref[0]
    spm = segpm_ref[0]
    ss = segs_ref[0]
    d = sp.shape[1]
    mean = spm * jnp.reciprocal(sp)
    pm_out[0] = mean
    seg_pmm = ss[:, 0:1]
    seg_rlp = ss[:, 1:2]
    ns = jnp.maximum(ss[:, 2:3], 1.0)
    expo = 0.5 * (jnp.sum(spm * mean, axis=1, keepdims=True) - seg_pmm)
    log_det = 0.5 * (seg_rlp - jnp.sum(jnp.log(sp), axis=1, keepdims=True))
    ln = 0.5 * (1.0 - ns) * (d * LOG_2PI) + log_det + expo
    ln_out[0] = ln.reshape(1, NUM_CLASSES)


def kernel(means, precisions, targets):
    b, n, d = means.shape
    e = b * n
    rows = b * NUM_CLASSES
    dh = d // NC
    t3 = targets.reshape(b, 1, n)

    pm_full, srows, idx = pl.pallas_call(
        _pre_body,
        grid=(b,),
        in_specs=[
            pl.BlockSpec((1, 1, n), lambda i: (i, 0, 0)),
            pl.BlockSpec((1, n, d), lambda i: (i, 0, 0)),
            pl.BlockSpec((1, n, d), lambda i: (i, 0, 0)),
        ],
        out_specs=[
            pl.BlockSpec((1, n, d), lambda i: (i, 0, 0)),
            pl.BlockSpec((1, n, 16), lambda i: (i, 0, 0)),
            pl.BlockSpec((1, 1, n), lambda i: (i, 0, 0)),
        ],
        out_shape=[
            jax.ShapeDtypeStruct((b, n, d), jnp.float32),
            jax.ShapeDtypeStruct((b, n, 16), jnp.float32),
            jax.ShapeDtypeStruct((b, 1, n), jnp.int32),
        ],
    )(t3, precisions, means)

    p_flat = precisions.reshape(e, d)
    pm_flat = pm_full.reshape(e, d)
    s_flat = srows.reshape(e, 16)
    idx_flat = idx.reshape(e)
    dq = d // (2 * NC)
    z = jnp.zeros((rows // NS, dq), jnp.float32)
    zs = jnp.zeros((rows // NS, 16), jnp.float32)

    sc_scatter = pl.kernel(
        _sc_body,
        out_type=[
            jax.ShapeDtypeStruct((rows, d), jnp.float32),
            jax.ShapeDtypeStruct((rows, d), jnp.float32),
            jax.ShapeDtypeStruct((rows, 16), jnp.float32),
        ],
        mesh=plsc.VectorSubcoreMesh(core_axis_name="c", subcore_axis_name="s"),
        scratch_types=[
            pltpu.VMEM((CH, dq), jnp.float32),
            pltpu.VMEM((CH, dq), jnp.float32),
            pltpu.VMEM((CH, dq), jnp.float32),
            pltpu.VMEM((CH, dq), jnp.float32),
            pltpu.VMEM((CH, 16), jnp.float32),
            pltpu.VMEM((CH, 16), jnp.float32),
            pltpu.VMEM((CH,), jnp.int32),
            pltpu.VMEM((CH,), jnp.int32),
            pltpu.VMEM_SHARED((rows, dq), jnp.float32),
            pltpu.VMEM_SHARED((rows, dq), jnp.float32),
            pltpu.VMEM_SHARED((rows, 16), jnp.float32),
            pltpu.SemaphoreType.DMA,
        ],
    )
    segp, segpm, segs = sc_scatter(p_flat, pm_flat, s_flat, idx_flat, z, zs)

    pm_o, ln_o = pl.pallas_call(
        _fin_body,
        grid=(b,),
        in_specs=[
            pl.BlockSpec((1, NUM_CLASSES, d), lambda i: (i, 0, 0)),
            pl.BlockSpec((1, NUM_CLASSES, d), lambda i: (i, 0, 0)),
            pl.BlockSpec((1, NUM_CLASSES, 16), lambda i: (i, 0, 0)),
        ],
        out_specs=[
            pl.BlockSpec((1, NUM_CLASSES, d), lambda i: (i, 0, 0)),
            pl.BlockSpec((1, 1, NUM_CLASSES), lambda i: (i, 0, 0)),
        ],
        out_shape=[
            jax.ShapeDtypeStruct((b, NUM_CLASSES, d), jnp.float32),
            jax.ShapeDtypeStruct((b, 1, NUM_CLASSES), jnp.float32),
        ],
    )(segp.reshape(b, NUM_CLASSES, d),
      segpm.reshape(b, NUM_CLASSES, d),
      segs.reshape(b, NUM_CLASSES, 16))

    return (pm_o,
            segp.reshape(b, NUM_CLASSES, d),
            ln_o.reshape(b, NUM_CLASSES))


# speedup vs baseline: 1.0895x; 1.0895x over previous
import functools
import math

import jax
import jax.numpy as jnp
from jax import lax
from jax.experimental import pallas as pl
from jax.experimental.pallas import tpu as pltpu
from jax.experimental.pallas import tpu_sc as plsc

NUM_CLASSES = 128
LOG_2PI = math.log(2.0 * math.pi)

NC = 2
NS = 16
CH = 64


def _pre_body(t_ref, p_ref, m_ref, pm_out, s_out, idx_out):
    p = p_ref[0]
    m = m_ref[0]
    n_ex, _ = p.shape
    pm = p * m
    pm_out[0] = pm
    r_pmm = jnp.sum(pm * m, axis=1, keepdims=True)
    r_lp = jnp.sum(jnp.log(p), axis=1, keepdims=True)
    ones = jnp.ones((n_ex, 1), jnp.float32)
    zeros = jnp.zeros((n_ex, 13), jnp.float32)
    s_out[0] = jnp.concatenate([r_pmm, r_lp, ones, zeros], axis=1)
    idx_out[0] = t_ref[0] + pl.program_id(0) * NUM_CLASSES


def _sc_body(p_hbm, pm_hbm, s_hbm, idx_hbm, z_hbm, zs_hbm,
             segp_hbm, segpm_hbm, segs_hbm,
             pbuf0, pbuf1, pmbuf0, pmbuf1, sbuf0, sbuf1, idx0, idx1,
             accp, accpm, accs, sem, sem2):
    cid = lax.axis_index("c")
    sid = lax.axis_index("s")
    n_rows = segp_hbm.shape[0]
    rows_t = n_rows // NS
    d = p_hbm.shape[1]
    dq = d // (2 * NC)
    ep_t = p_hbm.shape[0] // NS
    nchunk = ep_t // CH
    r0 = sid * rows_t

    pb = (pbuf0, pbuf1)
    pmb = (pmbuf0, pmbuf1)
    sb = (sbuf0, sbuf1)
    ib = (idx0, idx1)

    for q in range(2):
        c0 = (q * NC + cid) * dq
        first = q == 0

        pltpu.sync_copy(z_hbm, accp.at[pl.ds(r0, rows_t)])
        pltpu.sync_copy(z_hbm, accpm.at[pl.ds(r0, rows_t)])
        if first:
            @pl.when(cid == 0)
            def _():
                pltpu.sync_copy(zs_hbm, accs.at[pl.ds(r0, rows_t)])

        plsc.subcore_barrier()

        def fire_gather(g, first=first):
            slot = g % 2
            e0 = sid * ep_t + g * CH
            cps = [
                pltpu.async_copy(idx_hbm.at[pl.ds(e0, CH)], ib[slot], sem),
                pltpu.async_copy(
                    p_hbm.at[pl.ds(e0, CH), pl.ds(c0, dq)], pb[slot], sem),
                pltpu.async_copy(
                    pm_hbm.at[pl.ds(e0, CH), pl.ds(c0, dq)], pmb[slot], sem),
            ]
            if first:
                cps.append(
                    pltpu.async_copy(s_hbm.at[pl.ds(e0, CH)], sb[slot], sem))
            return cps

        def fire_scatter(g, first=first):
            slot = g % 2
            cps = [
                pltpu.async_copy(pb[slot], accp.at[ib[slot]], sem2, add=True),
                pltpu.async_copy(pmb[slot], accpm.at[ib[slot]], sem2,
                                 add=True),
            ]
            if first:
                @pl.when(cid == 0)
                def _():
                    pltpu.sync_copy(sb[slot], accs.at[ib[slot]], add=True)
            return cps

        gat = {0: fire_gather(0)}
        if nchunk > 1:
            gat[1] = fire_gather(1)
        tail = []
        for g in range(nchunk):
            for cp in gat.pop(g):
                cp.wait()
            sc = fire_scatter(g)
            if g + 2 < nchunk:
                for cp in sc:
                    cp.wait()
                gat[g + 2] = fire_gather(g + 2)
            else:
                tail.append(sc)
        for sc in tail:
            for cp in sc:
                cp.wait()

        plsc.subcore_barrier()

        pltpu.sync_copy(accp.at[pl.ds(r0, rows_t)],
                        segp_hbm.at[pl.ds(r0, rows_t), pl.ds(c0, dq)])
        pltpu.sync_copy(accpm.at[pl.ds(r0, rows_t)],
                        segpm_hbm.at[pl.ds(r0, rows_t), pl.ds(c0, dq)])
        if first:
            @pl.when(cid == 0)
            def _():
                pltpu.sync_copy(accs.at[pl.ds(r0, rows_t)],
                                segs_hbm.at[pl.ds(r0, rows_t)])


def _fin_body(segp_ref, segpm_ref, segs_ref, pm_out, ln_out):
    sp = segp_ref[0]
    spm = segpm_ref[0]
    ss = segs_ref[0]
    d = sp.shape[1]
    mean = spm * jnp.reciprocal(sp)
    pm_out[0] = mean
    seg_pmm = ss[:, 0:1]
    seg_rlp = ss[:, 1:2]
    ns = jnp.maximum(ss[:, 2:3], 1.0)
    expo = 0.5 * (jnp.sum(spm * mean, axis=1, keepdims=True) - seg_pmm)
    log_det = 0.5 * (seg_rlp - jnp.sum(jnp.log(sp), axis=1, keepdims=True))
    ln = 0.5 * (1.0 - ns) * (d * LOG_2PI) + log_det + expo
    ln_out[0] = ln.reshape(1, NUM_CLASSES)


def kernel(means, precisions, targets):
    b, n, d = means.shape
    e = b * n
    rows = b * NUM_CLASSES
    dh = d // NC
    t3 = targets.reshape(b, 1, n)

    pm_full, srows, idx = pl.pallas_call(
        _pre_body,
        grid=(b,),
        in_specs=[
            pl.BlockSpec((1, 1, n), lambda i: (i, 0, 0)),
            pl.BlockSpec((1, n, d), lambda i: (i, 0, 0)),
            pl.BlockSpec((1, n, d), lambda i: (i, 0, 0)),
        ],
        out_specs=[
            pl.BlockSpec((1, n, d), lambda i: (i, 0, 0)),
            pl.BlockSpec((1, n, 16), lambda i: (i, 0, 0)),
            pl.BlockSpec((1, 1, n), lambda i: (i, 0, 0)),
        ],
        out_shape=[
            jax.ShapeDtypeStruct((b, n, d), jnp.float32),
            jax.ShapeDtypeStruct((b, n, 16), jnp.float32),
            jax.ShapeDtypeStruct((b, 1, n), jnp.int32),
        ],
    )(t3, precisions, means)

    p_flat = precisions.reshape(e, d)
    pm_flat = pm_full.reshape(e, d)
    s_flat = srows.reshape(e, 16)
    idx_flat = idx.reshape(e)
    dq = d // (2 * NC)
    z = jnp.zeros((rows // NS, dq), jnp.float32)
    zs = jnp.zeros((rows // NS, 16), jnp.float32)

    sc_scatter = pl.kernel(
        _sc_body,
        out_type=[
            jax.ShapeDtypeStruct((rows, d), jnp.float32),
            jax.ShapeDtypeStruct((rows, d), jnp.float32),
            jax.ShapeDtypeStruct((rows, 16), jnp.float32),
        ],
        mesh=plsc.VectorSubcoreMesh(core_axis_name="c", subcore_axis_name="s"),
        scratch_types=[
            pltpu.VMEM((CH, dq), jnp.float32),
            pltpu.VMEM((CH, dq), jnp.float32),
            pltpu.VMEM((CH, dq), jnp.float32),
            pltpu.VMEM((CH, dq), jnp.float32),
            pltpu.VMEM((CH, 16), jnp.float32),
            pltpu.VMEM((CH, 16), jnp.float32),
            pltpu.VMEM((CH,), jnp.int32),
            pltpu.VMEM((CH,), jnp.int32),
            pltpu.VMEM_SHARED((rows, dq), jnp.float32),
            pltpu.VMEM_SHARED((rows, dq), jnp.float32),
            pltpu.VMEM_SHARED((rows, 16), jnp.float32),
            pltpu.SemaphoreType.DMA,
            pltpu.SemaphoreType.DMA,
        ],
    )
    segp, segpm, segs = sc_scatter(p_flat, pm_flat, s_flat, idx_flat, z, zs)

    pm_o, ln_o = pl.pallas_call(
        _fin_body,
        grid=(b,),
        in_specs=[
            pl.BlockSpec((1, NUM_CLASSES, d), lambda i: (i, 0, 0)),
            pl.BlockSpec((1, NUM_CLASSES, d), lambda i: (i, 0, 0)),
            pl.BlockSpec((1, NUM_CLASSES, 16), lambda i: (i, 0, 0)),
        ],
        out_specs=[
            pl.BlockSpec((1, NUM_CLASSES, d), lambda i: (i, 0, 0)),
            pl.BlockSpec((1, 1, NUM_CLASSES), lambda i: (i, 0, 0)),
        ],
        out_shape=[
            jax.ShapeDtypeStruct((b, NUM_CLASSES, d), jnp.float32),
            jax.ShapeDtypeStruct((b, 1, NUM_CLASSES), jnp.float32),
        ],
    )(segp.reshape(b, NUM_CLASSES, d),
      segpm.reshape(b, NUM_CLASSES, d),
      segs.reshape(b, NUM_CLASSES, 16))

    return (pm_o,
            segp.reshape(b, NUM_CLASSES, d),
            ln_o.reshape(b, NUM_CLASSES))


# speedup vs baseline: 2.1603x; 1.9828x over previous
import math

import jax
import jax.numpy as jnp
from jax import lax
from jax.experimental import pallas as pl
from jax.experimental.pallas import tpu as pltpu
from jax.experimental.pallas import tpu_sc as plsc

NUM_CLASSES = 128
LOG_2PI = math.log(2.0 * math.pi)

NC = 2
NS = 16
CH = 64


def _scp_body(p_hbm, t_hbm, z_hbm, segp_hbm,
              pbufa0, pbufa1, pbufb0, pbufb1, tbuf0, tbuf1,
              acca, accb, sem, sem2):
    cid = lax.axis_index("c")
    sid = lax.axis_index("s")
    d = p_hbm.shape[1]
    dq = d // (2 * NC)
    ep_t = p_hbm.shape[0] // NS
    nchunk = ep_t // CH
    c0a = cid * dq
    c0b = (NC + cid) * dq
    r0 = sid * NUM_CLASSES

    pltpu.sync_copy(z_hbm, acca.at[pl.ds(r0, NUM_CLASSES)])
    pltpu.sync_copy(z_hbm, accb.at[pl.ds(r0, NUM_CLASSES)])

    pba = (pbufa0, pbufa1)
    pbb = (pbufb0, pbufb1)
    tb = (tbuf0, tbuf1)

    def fire_gather(g):
        slot = g % 2
        e0 = sid * ep_t + g * CH
        return [
            pltpu.async_copy(t_hbm.at[pl.ds(e0, CH)], tb[slot], sem),
            pltpu.async_copy(
                p_hbm.at[pl.ds(e0, CH), pl.ds(c0a, dq)], pba[slot], sem),
            pltpu.async_copy(
                p_hbm.at[pl.ds(e0, CH), pl.ds(c0b, dq)], pbb[slot], sem),
        ]

    def fire_scatter(g):
        slot = g % 2
        return [
            pltpu.async_copy(pba[slot], acca.at[tb[slot]], sem2, add=True),
            pltpu.async_copy(pbb[slot], accb.at[tb[slot]], sem2, add=True),
        ]

    gat = {0: fire_gather(0), 1: fire_gather(1)}
    tail = []
    for g in range(nchunk):
        for cp in gat.pop(g):
            cp.wait()
        sc = fire_scatter(g)
        if g + 2 < nchunk:
            for cp in sc:
                cp.wait()
            gat[g + 2] = fire_gather(g + 2)
        else:
            tail.append(sc)
    for sc in tail:
        for cp in sc:
            cp.wait()

    pltpu.sync_copy(acca.at[pl.ds(r0, NUM_CLASSES)],
                    segp_hbm.at[pl.ds(r0, NUM_CLASSES), pl.ds(c0a, dq)])
    pltpu.sync_copy(accb.at[pl.ds(r0, NUM_CLASSES)],
                    segp_hbm.at[pl.ds(r0, NUM_CLASSES), pl.ds(c0b, dq)])


def _split_dot(oh, x):
    x_hi = x.astype(jnp.bfloat16)
    x_lo = (x - x_hi.astype(jnp.float32)).astype(jnp.bfloat16)
    dn = (((0,), (0,)), ((), ()))
    hi = lax.dot_general(oh, x_hi, dn, preferred_element_type=jnp.float32)
    lo = lax.dot_general(oh, x_lo, dn, preferred_element_type=jnp.float32)
    return hi + lo


def _tc_body(t_ref, p_ref, m_ref, segpm_out, scal_out):
    p = p_ref[0]
    m = m_ref[0]
    t = t_ref[0]
    n_ex, _ = p.shape
    cls = lax.broadcasted_iota(jnp.int32, (n_ex, NUM_CLASSES), 1)
    mask = t.reshape(n_ex, 1) == cls
    oh = mask.astype(jnp.bfloat16)

    pm = p * m
    segpm_out[0] = _split_dot(oh, pm)

    r_pmm = jnp.sum(pm * m, axis=1, keepdims=True)
    r_lp = jnp.sum(jnp.log(p), axis=1, keepdims=True)
    seg_pmm = jnp.sum(jnp.where(mask, r_pmm, 0.0), axis=0, keepdims=True)
    seg_rlp = jnp.sum(jnp.where(mask, r_lp, 0.0), axis=0, keepdims=True)
    counts = jnp.sum(mask.astype(jnp.float32), axis=0, keepdims=True)
    zero5 = jnp.zeros((5, NUM_CLASSES), jnp.float32)
    scal_out[0] = jnp.concatenate([seg_pmm, seg_rlp, counts, zero5], axis=0)


def _fin_body(segp_ref, segpm_ref, scal_ref, pm_out, ln_out):
    sp = segp_ref[0]
    spm = segpm_ref[0]
    sc = scal_ref[0]
    d = sp.shape[1]
    mean = spm * jnp.reciprocal(sp)
    pm_out[0] = mean
    seg_pmm = sc[0:1, :]
    seg_rlp = sc[1:2, :]
    ns = jnp.maximum(sc[2:3, :], 1.0)
    expo = 0.5 * (jnp.sum(spm * mean, axis=1).reshape(1, NUM_CLASSES)
                  - seg_pmm)
    log_det = 0.5 * (seg_rlp
                     - jnp.sum(jnp.log(sp), axis=1).reshape(1, NUM_CLASSES))
    ln_out[0] = 0.5 * (1.0 - ns) * (d * LOG_2PI) + log_det + expo


def kernel(means, precisions, targets):
    b, n, d = means.shape
    e = b * n
    rows = b * NUM_CLASSES
    dh = d // NC
    t3 = targets.reshape(b, 1, n)
    rowid = (targets + NUM_CLASSES * jnp.arange(b, dtype=jnp.int32)[:, None]
             ).reshape(e)
    p_flat = precisions.reshape(e, d)
    dq = d // (2 * NC)
    z = jnp.zeros((NUM_CLASSES, dq), jnp.float32)

    sc_scatter = pl.kernel(
        _scp_body,
        out_type=[jax.ShapeDtypeStruct((rows, d), jnp.float32)],
        mesh=plsc.VectorSubcoreMesh(core_axis_name="c", subcore_axis_name="s"),
        scratch_types=[
            pltpu.VMEM((CH, dq), jnp.float32),
            pltpu.VMEM((CH, dq), jnp.float32),
            pltpu.VMEM((CH, dq), jnp.float32),
            pltpu.VMEM((CH, dq), jnp.float32),
            pltpu.VMEM((CH,), jnp.int32),
            pltpu.VMEM((CH,), jnp.int32),
            pltpu.VMEM_SHARED((rows, dq), jnp.float32),
            pltpu.VMEM_SHARED((rows, dq), jnp.float32),
            pltpu.SemaphoreType.DMA,
            pltpu.SemaphoreType.DMA,
        ],
    )
    (segp,) = sc_scatter(p_flat, rowid, z)

    segpm, scal = pl.pallas_call(
        _tc_body,
        grid=(b,),
        in_specs=[
            pl.BlockSpec((1, 1, n), lambda i: (i, 0, 0)),
            pl.BlockSpec((1, n, d), lambda i: (i, 0, 0)),
            pl.BlockSpec((1, n, d), lambda i: (i, 0, 0)),
        ],
        out_specs=[
            pl.BlockSpec((1, NUM_CLASSES, d), lambda i: (i, 0, 0)),
            pl.BlockSpec((1, 8, NUM_CLASSES), lambda i: (i, 0, 0)),
        ],
        out_shape=[
            jax.ShapeDtypeStruct((b, NUM_CLASSES, d), jnp.float32),
            jax.ShapeDtypeStruct((b, 8, NUM_CLASSES), jnp.float32),
        ],
    )(t3, precisions, means)

    segp3 = segp.reshape(b, NUM_CLASSES, d)
    pm_o, ln_o = pl.pallas_call(
        _fin_body,
        grid=(b,),
        in_specs=[
            pl.BlockSpec((1, NUM_CLASSES, d), lambda i: (i, 0, 0)),
            pl.BlockSpec((1, NUM_CLASSES, d), lambda i: (i, 0, 0)),
            pl.BlockSpec((1, 8, NUM_CLASSES), lambda i: (i, 0, 0)),
        ],
        out_specs=[
            pl.BlockSpec((1, NUM_CLASSES, d), lambda i: (i, 0, 0)),
            pl.BlockSpec((1, 1, NUM_CLASSES), lambda i: (i, 0, 0)),
        ],
        out_shape=[
            jax.ShapeDtypeStruct((b, NUM_CLASSES, d), jnp.float32),
            jax.ShapeDtypeStruct((b, 1, NUM_CLASSES), jnp.float32),
        ],
    )(segp3, segpm, scal)

    return (pm_o, segp3, ln_o.reshape(b, NUM_CLASSES))


# speedup vs baseline: 2.1748x; 1.0067x over previous
import math

import jax
import jax.numpy as jnp
from jax import lax
from jax.experimental import pallas as pl
from jax.experimental.pallas import tpu as pltpu
from jax.experimental.pallas import tpu_sc as plsc

NUM_CLASSES = 128
LOG_2PI = math.log(2.0 * math.pi)

NC = 2
NS = 16
CH = 128


def _scp_body(p_hbm, idx_hbm, z_hbm, segp_hbm,
              pbufa0, pbufa1, pbufb, idxv, acca, accb, sem, sem2):
    cid = lax.axis_index("c")
    sid = lax.axis_index("s")
    d = p_hbm.shape[1]
    dq = d // (2 * NC)
    ep_t = p_hbm.shape[0] // NS
    nchunk = ep_t // CH
    c0a = cid * dq
    c0b = (NC + cid) * dq
    r0 = sid * NUM_CLASSES

    pltpu.sync_copy(idx_hbm.at[pl.ds(sid * nchunk, nchunk)], idxv)
    pltpu.sync_copy(z_hbm, acca.at[pl.ds(r0, NUM_CLASSES)])
    pltpu.sync_copy(z_hbm, accb.at[pl.ds(r0, NUM_CLASSES)])

    pba = (pbufa0, pbufa1)

    def gather(buf, g, c0):
        e0 = sid * ep_t + g * CH
        return pltpu.async_copy(
            p_hbm.at[pl.ds(e0, CH), pl.ds(c0, dq)], buf, sem)

    def scatter(buf, g, acc):
        return pltpu.async_copy(buf, acc.at[idxv.at[g]], sem2, add=True)

    ga = {0: gather(pba[0], 0, c0a), 1: gather(pba[1], 1, c0a)}
    gb = {0: gather(pbufb, 0, c0b)}
    tail = []
    for g in range(nchunk):
        ga.pop(g).wait()
        gb.pop(g).wait()
        sca = scatter(pba[g % 2], g, acca)
        scb = scatter(pbufb, g, accb)
        scb.wait()
        if g + 1 < nchunk:
            gb[g + 1] = gather(pbufb, g + 1, c0b)
        if g + 2 < nchunk:
            sca.wait()
            ga[g + 2] = gather(pba[g % 2], g + 2, c0a)
        else:
            tail.append(sca)
    for sca in tail:
        sca.wait()

    pltpu.sync_copy(acca.at[pl.ds(r0, NUM_CLASSES)],
                    segp_hbm.at[pl.ds(r0, NUM_CLASSES), pl.ds(c0a, dq)])
    pltpu.sync_copy(accb.at[pl.ds(r0, NUM_CLASSES)],
                    segp_hbm.at[pl.ds(r0, NUM_CLASSES), pl.ds(c0b, dq)])


def _split_dot(oh, x):
    x_hi = x.astype(jnp.bfloat16)
    x_lo = (x - x_hi.astype(jnp.float32)).astype(jnp.bfloat16)
    dn = (((0,), (0,)), ((), ()))
    hi = lax.dot_general(oh, x_hi, dn, preferred_element_type=jnp.float32)
    lo = lax.dot_general(oh, x_lo, dn, preferred_element_type=jnp.float32)
    return hi + lo


def _tc_body(t_ref, p_ref, m_ref, segpm_out, scal_out):
    p = p_ref[0]
    m = m_ref[0]
    t = t_ref[0]
    n_ex, _ = p.shape
    cls = lax.broadcasted_iota(jnp.int32, (n_ex, NUM_CLASSES), 1)
    mask = t.reshape(n_ex, 1) == cls
    oh = mask.astype(jnp.bfloat16)

    pm = p * m
    segpm_out[0] = _split_dot(oh, pm)

    r_pmm = jnp.sum(pm * m, axis=1, keepdims=True)
    r_lp = jnp.sum(jnp.log(p), axis=1, keepdims=True)
    seg_pmm = jnp.sum(jnp.where(mask, r_pmm, 0.0), axis=0, keepdims=True)
    seg_rlp = jnp.sum(jnp.where(mask, r_lp, 0.0), axis=0, keepdims=True)
    counts = jnp.sum(mask.astype(jnp.float32), axis=0, keepdims=True)
    zero5 = jnp.zeros((5, NUM_CLASSES), jnp.float32)
    scal_out[0] = jnp.concatenate([seg_pmm, seg_rlp, counts, zero5], axis=0)


def _fin_body(segp_ref, segpm_ref, scal_ref, pm_out, ln_out):
    sp = segp_ref[0]
    spm = segpm_ref[0]
    sc = scal_ref[0]
    d = sp.shape[1]
    mean = spm * jnp.reciprocal(sp)
    pm_out[0] = mean
    seg_pmm = sc[0:1, :]
    seg_rlp = sc[1:2, :]
    ns = jnp.maximum(sc[2:3, :], 1.0)
    expo = 0.5 * (jnp.sum(spm * mean, axis=1).reshape(1, NUM_CLASSES)
                  - seg_pmm)
    log_det = 0.5 * (seg_rlp
                     - jnp.sum(jnp.log(sp), axis=1).reshape(1, NUM_CLASSES))
    ln_out[0] = 0.5 * (1.0 - ns) * (d * LOG_2PI) + log_det + expo


def kernel(means, precisions, targets):
    b, n, d = means.shape
    e = b * n
    rows = b * NUM_CLASSES
    dh = d // NC
    t3 = targets.reshape(b, 1, n)
    rowid = (targets + NUM_CLASSES * jnp.arange(b, dtype=jnp.int32)[:, None]
             ).reshape(e // CH, CH)
    p_flat = precisions.reshape(e, d)
    dq = d // (2 * NC)
    nchunk = n // CH
    z = jnp.zeros((NUM_CLASSES, dq), jnp.float32)

    sc_scatter = pl.kernel(
        _scp_body,
        out_type=[jax.ShapeDtypeStruct((rows, d), jnp.float32)],
        mesh=plsc.VectorSubcoreMesh(core_axis_name="c", subcore_axis_name="s"),
        scratch_types=[
            pltpu.VMEM((CH, dq), jnp.float32),
            pltpu.VMEM((CH, dq), jnp.float32),
            pltpu.VMEM((CH, dq), jnp.float32),
            pltpu.VMEM((nchunk, CH), jnp.int32),
            pltpu.VMEM_SHARED((rows, dq), jnp.float32),
            pltpu.VMEM_SHARED((rows, dq), jnp.float32),
            pltpu.SemaphoreType.DMA,
            pltpu.SemaphoreType.DMA,
        ],
    )
    (segp,) = sc_scatter(p_flat, rowid, z)

    segpm, scal = pl.pallas_call(
        _tc_body,
        grid=(b,),
        in_specs=[
            pl.BlockSpec((1, 1, n), lambda i: (i, 0, 0)),
            pl.BlockSpec((1, n, d), lambda i: (i, 0, 0)),
            pl.BlockSpec((1, n, d), lambda i: (i, 0, 0)),
        ],
        out_specs=[
            pl.BlockSpec((1, NUM_CLASSES, d), lambda i: (i, 0, 0)),
            pl.BlockSpec((1, 8, NUM_CLASSES), lambda i: (i, 0, 0)),
        ],
        out_shape=[
            jax.ShapeDtypeStruct((b, NUM_CLASSES, d), jnp.float32),
            jax.ShapeDtypeStruct((b, 8, NUM_CLASSES), jnp.float32),
        ],
    )(t3, precisions, means)

    segp3 = segp.reshape(b, NUM_CLASSES, d)
    pm_o, ln_o = pl.pallas_call(
        _fin_body,
        grid=(b,),
        in_specs=[
            pl.BlockSpec((1, NUM_CLASSES, d), lambda i: (i, 0, 0)),
            pl.BlockSpec((1, NUM_CLASSES, d), lambda i: (i, 0, 0)),
            pl.BlockSpec((1, 8, NUM_CLASSES), lambda i: (i, 0, 0)),
        ],
        out_specs=[
            pl.BlockSpec((1, NUM_CLASSES, d), lambda i: (i, 0, 0)),
            pl.BlockSpec((1, 1, NUM_CLASSES), lambda i: (i, 0, 0)),
        ],
        out_shape=[
            jax.ShapeDtypeStruct((b, NUM_CLASSES, d), jnp.float32),
            jax.ShapeDtypeStruct((b, 1, NUM_CLASSES), jnp.float32),
        ],
    )(segp3, segpm, scal)

    return (pm_o, segp3, ln_o.reshape(b, NUM_CLASSES))
